# Initial kernel scaffold; baseline (speedup 1.0000x reference)
import jax
import jax.numpy as jnp
from jax.experimental import pallas as pl


def kernel(tokens, ws, ks, land_sea_mask, W1, b1, W2, b2):
    raise NotImplementedError("write your pallas kernel here")


# speedup vs baseline: 54.5724x; 54.5724x over previous
import functools

import jax
import jax.numpy as jnp
from jax.experimental import pallas as pl

B, N, D, E, H = 32, 32768, 4, 2, 32


def _select_kernel(ws_ref, ks_ref, b1_ref, w2_ref, b2_ref,
                   t_ref, c_ref, cvec_ref):
    u = jax.lax.bitcast_convert_type(ws_ref[...], jnp.int32)
    keys = u ^ ((u >> 31) & jnp.int32(0x7FFFFFFF))
    keys = jnp.where(keys == -1, 0, keys)
    k = jnp.clip(ks_ref[...], 1, N - 1)

    def t_body(t, cur):
        shift = jnp.int32(31) - t
        cand = cur + (jnp.int32(1) << shift)
        cnt = jnp.sum((keys >= cand).astype(jnp.int32), axis=1, keepdims=True)
        return jnp.where(cnt >= k, cand, cur)

    t_val = jax.lax.fori_loop(
        0, 32, t_body, jnp.full((B, 1), jnp.int32(-2147483648)))

    n_gt = jnp.sum((keys > t_val).astype(jnp.int32), axis=1, keepdims=True)
    m = k - n_gt
    eq = (keys == t_val)
    idx = jax.lax.broadcasted_iota(jnp.int32, (B, N), 1)

    def c_body(t, carry):
        lo, hi = carry
        mid = (lo + hi) >> 1
        c2 = jnp.sum((eq & (idx < mid)).astype(jnp.int32), axis=1,
                     keepdims=True)
        ok = c2 >= m
        return jnp.where(ok, lo, mid + 1), jnp.where(ok, mid, hi)

    lo, _ = jax.lax.fori_loop(
        0, 16, c_body,
        (jnp.zeros((B, 1), jnp.int32), jnp.full((B, 1), jnp.int32(N))))

    t_ref[...] = t_val
    c_ref[...] = lo
    h = jax.nn.gelu(b1_ref[...])
    cvec_ref[...] = jnp.sum(h[0, :, None] * w2_ref[...], axis=0,
                            keepdims=True) + b2_ref[...]


def _reduce_kernel(t0_ref, t1_ref, t2_ref, t3_ref, ws_ref, land_ref,
                   tk_ref, ci_ref, cvec_ref, s_ref, cnt_ref, *, n_blk):
    step = pl.program_id(0)

    u = jax.lax.bitcast_convert_type(ws_ref[...], jnp.int32)
    keys = u ^ ((u >> 31) & jnp.int32(0x7FFFFFFF))
    keys = jnp.where(keys == -1, 0, keys)
    gcol = (jax.lax.broadcasted_iota(jnp.int32, (B, n_blk), 1)
            + step * n_blk)
    t_val = tk_ref[...]
    vis = (keys > t_val) | ((keys == t_val) & (gcol < ci_ref[...]))
    maskf = jnp.where(land_ref[...] & ~vis, jnp.float32(1.0),
                      jnp.float32(0.0))

    cvec = cvec_ref[...]
    q = jnp.zeros((B, n_blk), jnp.float32)
    k2 = jnp.float32(0.0)
    for d, t_d in enumerate((t0_ref, t1_ref, t2_ref, t3_ref)):
        c0 = cvec[0:1, 2 * d:2 * d + 1]
        c1 = cvec[0:1, 2 * d + 1:2 * d + 2]
        t = t_d[...]
        q += 0.5 * (jnp.abs(c0 - t) + jnp.abs(c1 - t))
        k2 += 0.25 * jnp.abs(c0 - c1)[0, 0]

    @pl.when(step == 0)
    def _():
        s_ref[...] = jnp.zeros_like(s_ref)
        cnt_ref[...] = jnp.zeros_like(cnt_ref)

    pcnt = jnp.sum(maskf, axis=1, keepdims=True)
    s_ref[...] += jnp.sum(maskf * q, axis=1, keepdims=True) - k2 * pcnt
    cnt_ref[...] += pcnt


@jax.jit
def kernel(tokens, ws, ks, land_sea_mask, W1, b1, W2, b2):
    del W1
    t_sel, c_sel, cvec = pl.pallas_call(
        _select_kernel,
        out_shape=(
            jax.ShapeDtypeStruct((B, 1), jnp.int32),
            jax.ShapeDtypeStruct((B, 1), jnp.int32),
            jax.ShapeDtypeStruct((1, D * E), jnp.float32),
        ),
    )(ws, ks.reshape(B, 1), b1.reshape(1, H), W2, b2.reshape(1, D * E))

    n_chunks = 8
    n_blk = N // n_chunks
    col_spec = pl.BlockSpec((B, n_blk), lambda j: (0, j))
    full_spec = pl.BlockSpec((B, 1), lambda j: (0, 0))
    s_sum, cnt = pl.pallas_call(
        functools.partial(_reduce_kernel, n_blk=n_blk),
        grid=(n_chunks,),
        in_specs=[col_spec, col_spec, col_spec, col_spec, col_spec, col_spec,
                  full_spec, full_spec,
                  pl.BlockSpec((1, D * E), lambda j: (0, 0))],
        out_specs=(full_spec, full_spec),
        out_shape=(
            jax.ShapeDtypeStruct((B, 1), jnp.float32),
            jax.ShapeDtypeStruct((B, 1), jnp.float32),
        ),
    )(tokens[:, :, 0], tokens[:, :, 1], tokens[:, :, 2], tokens[:, :, 3],
      ws, land_sea_mask.reshape(B, N),
      t_sel, c_sel, cvec)

    return jnp.sum(s_sum / cnt) / (B * D)
